# Initial kernel scaffold; baseline (speedup 1.0000x reference)
import jax
import jax.numpy as jnp
from jax.experimental import pallas as pl


def kernel(src, aa_table, pos_table, gamma, beta):
    raise NotImplementedError("write your pallas kernel here")


# speedup vs baseline: 3.1894x; 3.1894x over previous
import functools

import jax
import jax.numpy as jnp
from jax import lax
from jax.experimental import pallas as pl
from jax.experimental.pallas import tpu as pltpu
from jax.experimental.pallas import tpu_sc as plsc

VOCAB = 25
MAX_POS = 256
HIDDEN = 128
EPS = 1e-12

ROWS = 4096
COLS = 200
TOKENS = ROWS * COLS


def _table_body(aa_ref, pos_ref, g_ref, b_ref, f_ref):
    aa = aa_ref[...]
    pos = pos_ref[...]
    e = aa[:, None, :] + pos[None, :, :]
    mean = jnp.mean(e, axis=-1, keepdims=True)
    c = e - mean
    var = jnp.mean(c * c, axis=-1, keepdims=True)
    y = c * lax.rsqrt(var + EPS)
    y = y * g_ref[...][None, None, :] + b_ref[...][None, None, :]
    f_ref[...] = y


def _build_table(aa, pos, gamma, beta):
    return pl.pallas_call(
        _table_body,
        out_shape=jax.ShapeDtypeStruct((VOCAB, MAX_POS, HIDDEN), jnp.float32),
    )(aa, pos, gamma, beta)


def _fidx_body(src_ref, out_ref):
    s = src_ref[...]
    m = (s != 0).astype(jnp.int32)
    mb = m.astype(jnp.bfloat16)
    tri = (
        lax.broadcasted_iota(jnp.int32, (COLS, COLS), 0)
        <= lax.broadcasted_iota(jnp.int32, (COLS, COLS), 1)
    ).astype(jnp.bfloat16)
    cs = lax.dot_general(
        mb, tri, (((1,), (0,)), ((), ())),
        preferred_element_type=jnp.float32,
    )
    pid = cs.astype(jnp.int32) * m
    out_ref[...] = s * MAX_POS + pid


def _build_fidx(src):
    return pl.pallas_call(
        _fidx_body,
        out_shape=jax.ShapeDtypeStruct((ROWS, COLS), jnp.int32),
    )(src)


try:
    _INFO = plsc.get_sparse_core_info()
    _NC, _NS = _INFO.num_cores, _INFO.num_subcores
except ValueError:
    _NC, _NS = 2, 16
_NW = _NC * _NS
_CHUNK = 128
_PER_W = TOKENS // _NW
_G = _PER_W // _CHUNK


def _sc_gather(fidx3, table2):
    mesh = plsc.VectorSubcoreMesh(core_axis_name="c", subcore_axis_name="s")

    @functools.partial(
        pl.kernel,
        mesh=mesh,
        out_type=jax.ShapeDtypeStruct((TOKENS, HIDDEN), jnp.float32),
        scratch_types=[
            pltpu.VMEM((_G, _CHUNK), jnp.int32),
            pltpu.VMEM((_CHUNK, HIDDEN), jnp.float32),
            pltpu.VMEM((_CHUNK, HIDDEN), jnp.float32),
            pltpu.SemaphoreType.DMA,
            pltpu.SemaphoreType.DMA,
        ],
    )
    def k(fidx_hbm, f_hbm, out_hbm, idx_v, buf0, buf1, sem0, sem1):
        wid = lax.axis_index("s") * _NC + lax.axis_index("c")
        base = wid * _PER_W
        pltpu.sync_copy(fidx_hbm.at[wid], idx_v)

        def start(g, buf, sem):
            pltpu.async_copy(f_hbm.at[idx_v.at[g]], buf, sem)

        def wait(buf, sem):
            pltpu.make_async_copy(f_hbm.at[idx_v.at[0]], buf, sem).wait()

        def store(g, buf):
            pltpu.sync_copy(buf, out_hbm.at[pl.ds(base + g * _CHUNK, _CHUNK)])

        start(0, buf0, sem0)

        def body(h, carry):
            g0 = h * 2
            start(g0 + 1, buf1, sem1)
            wait(buf0, sem0)
            store(g0, buf0)

            @pl.when(g0 + 2 < _G)
            def _():
                start(g0 + 2, buf0, sem0)

            wait(buf1, sem1)
            store(g0 + 1, buf1)
            return carry

        lax.fori_loop(0, _G // 2, body, 0)

    return k(fidx3, table2)


def kernel(src, aa_table, pos_table, gamma, beta):
    table = _build_table(aa_table, pos_table, gamma, beta)
    fidx = _build_fidx(src)
    table2 = table.reshape(VOCAB * MAX_POS, HIDDEN)
    fidx3 = fidx.reshape(_NW, _G, _CHUNK)
    out = _sc_gather(fidx3, table2)
    return out.reshape(ROWS, COLS, HIDDEN)


# speedup vs baseline: 33.2390x; 10.4216x over previous
import functools

import jax
import jax.numpy as jnp
from jax import lax
from jax.experimental import pallas as pl
from jax.experimental.pallas import tpu as pltpu
from jax.experimental.pallas import tpu_sc as plsc

VOCAB = 25
MAX_POS = 256
HIDDEN = 128
EPS = 1e-12

ROWS = 4096
COLS = 200
TOKENS = ROWS * COLS


def _table_body(aa_ref, pos_ref, g_ref, b_ref, f_ref):
    aa = aa_ref[...]
    pos = pos_ref[...]
    e = aa[:, None, :] + pos[None, :, :]
    mean = jnp.mean(e, axis=-1, keepdims=True)
    c = e - mean
    var = jnp.mean(c * c, axis=-1, keepdims=True)
    y = c * lax.rsqrt(var + EPS)
    y = y * g_ref[...][None, None, :] + b_ref[...][None, None, :]
    f_ref[...] = y


def _build_table(aa, pos, gamma, beta):
    return pl.pallas_call(
        _table_body,
        out_shape=jax.ShapeDtypeStruct((VOCAB, MAX_POS, HIDDEN), jnp.float32),
    )(aa, pos, gamma, beta)


def _fidx_body(src_ref, out_ref):
    s = src_ref[...]
    m = (s != 0).astype(jnp.int32)
    mb = m.astype(jnp.bfloat16)
    tri = (
        lax.broadcasted_iota(jnp.int32, (COLS, COLS), 0)
        <= lax.broadcasted_iota(jnp.int32, (COLS, COLS), 1)
    ).astype(jnp.bfloat16)
    cs = lax.dot_general(
        mb, tri, (((1,), (0,)), ((), ())),
        preferred_element_type=jnp.float32,
    )
    pid = cs.astype(jnp.int32) * m
    out_ref[...] = s * MAX_POS + pid


def _build_fidx(src):
    return pl.pallas_call(
        _fidx_body,
        out_shape=jax.ShapeDtypeStruct((ROWS, COLS), jnp.int32),
    )(src)


try:
    _INFO = plsc.get_sparse_core_info()
    _NC, _NS = _INFO.num_cores, _INFO.num_subcores
except ValueError:
    _NC, _NS = 2, 16
_NW = _NC * _NS
_CHUNK = 128
_PER_W = TOKENS // _NW
_G = _PER_W // _CHUNK


def _sc_gather(fidx3, table2):
    mesh = plsc.VectorSubcoreMesh(core_axis_name="c", subcore_axis_name="s")

    @functools.partial(
        pl.kernel,
        mesh=mesh,
        out_type=jax.ShapeDtypeStruct((TOKENS, HIDDEN), jnp.float32),
        scratch_types=[
            pltpu.VMEM((_G, _CHUNK), jnp.int32),
            pltpu.VMEM((_CHUNK, HIDDEN), jnp.float32),
            pltpu.VMEM((_CHUNK, HIDDEN), jnp.float32),
            pltpu.VMEM_SHARED((VOCAB * MAX_POS, HIDDEN), jnp.float32),
            pltpu.SemaphoreType.DMA,
            pltpu.SemaphoreType.DMA,
        ],
    )
    def k(fidx_hbm, f_hbm, out_hbm, idx_v, buf0, buf1, f_sh, sem0, sem1):
        sid = lax.axis_index("s")
        wid = sid * _NC + lax.axis_index("c")
        base = wid * _PER_W

        @pl.when(sid == 0)
        def _():
            pltpu.sync_copy(f_hbm, f_sh)

        pltpu.sync_copy(fidx_hbm.at[wid], idx_v)
        plsc.subcore_barrier()

        def start(g, buf, sem):
            pltpu.async_copy(f_sh.at[idx_v.at[g]], buf, sem)

        def wait(buf, sem):
            pltpu.make_async_copy(f_sh.at[idx_v.at[0]], buf, sem).wait()

        def store(g, buf):
            pltpu.sync_copy(buf, out_hbm.at[pl.ds(base + g * _CHUNK, _CHUNK)])

        start(0, buf0, sem0)

        def body(h, carry):
            g0 = h * 2
            start(g0 + 1, buf1, sem1)
            wait(buf0, sem0)
            store(g0, buf0)

            @pl.when(g0 + 2 < _G)
            def _():
                start(g0 + 2, buf0, sem0)

            wait(buf1, sem1)
            store(g0 + 1, buf1)
            return carry

        lax.fori_loop(0, _G // 2, body, 0)

    return k(fidx3, table2)


def kernel(src, aa_table, pos_table, gamma, beta):
    table = _build_table(aa_table, pos_table, gamma, beta)
    fidx = _build_fidx(src)
    table2 = table.reshape(VOCAB * MAX_POS, HIDDEN)
    fidx3 = fidx.reshape(_NW, _G, _CHUNK)
    out = _sc_gather(fidx3, table2)
    return out.reshape(ROWS, COLS, HIDDEN)


# speedup vs baseline: 33.3231x; 1.0025x over previous
import functools

import jax
import jax.numpy as jnp
from jax import lax
from jax.experimental import pallas as pl
from jax.experimental.pallas import tpu as pltpu
from jax.experimental.pallas import tpu_sc as plsc

VOCAB = 25
MAX_POS = 256
HIDDEN = 128
EPS = 1e-12

ROWS = 4096
COLS = 200
TOKENS = ROWS * COLS


def _table_body(aa_ref, pos_ref, g_ref, b_ref, f_ref):
    aa = aa_ref[...]
    pos = pos_ref[...]
    e = aa[:, None, :] + pos[None, :, :]
    mean = jnp.mean(e, axis=-1, keepdims=True)
    c = e - mean
    var = jnp.mean(c * c, axis=-1, keepdims=True)
    y = c * lax.rsqrt(var + EPS)
    y = y * g_ref[...][None, None, :] + b_ref[...][None, None, :]
    f_ref[...] = y


def _build_table(aa, pos, gamma, beta):
    return pl.pallas_call(
        _table_body,
        out_shape=jax.ShapeDtypeStruct((VOCAB, MAX_POS, HIDDEN), jnp.float32),
    )(aa, pos, gamma, beta)


def _fidx_body(src_ref, out_ref):
    s = src_ref[...]
    m = (s != 0).astype(jnp.int32)
    mb = m.astype(jnp.bfloat16)
    tri = (
        lax.broadcasted_iota(jnp.int32, (COLS, COLS), 0)
        <= lax.broadcasted_iota(jnp.int32, (COLS, COLS), 1)
    ).astype(jnp.bfloat16)
    cs = lax.dot_general(
        mb, tri, (((1,), (0,)), ((), ())),
        preferred_element_type=jnp.float32,
    )
    pid = cs.astype(jnp.int32) * m
    out_ref[...] = s * MAX_POS + pid


def _build_fidx(src):
    return pl.pallas_call(
        _fidx_body,
        out_shape=jax.ShapeDtypeStruct((ROWS, COLS), jnp.int32),
    )(src)


try:
    _INFO = plsc.get_sparse_core_info()
    _NC, _NS = _INFO.num_cores, _INFO.num_subcores
except ValueError:
    _NC, _NS = 2, 16
_NW = _NC * _NS
_CHUNK = 128
_PER_W = TOKENS // _NW
_G = _PER_W // _CHUNK
_STEP = _CHUNK
_H = _PER_W // _STEP


def _sc_gather(fidx3, table2):
    mesh = plsc.VectorSubcoreMesh(core_axis_name="c", subcore_axis_name="s")

    @functools.partial(
        pl.kernel,
        mesh=mesh,
        out_type=jax.ShapeDtypeStruct((TOKENS, HIDDEN), jnp.float32),
        scratch_types=[
            pltpu.VMEM((_G, _CHUNK), jnp.int32),
            pltpu.VMEM((_STEP, HIDDEN), jnp.float32),
            pltpu.VMEM((_STEP, HIDDEN), jnp.float32),
            pltpu.VMEM_SHARED((VOCAB * MAX_POS, HIDDEN), jnp.float32),
            pltpu.SemaphoreType.DMA,
            pltpu.SemaphoreType.DMA,
            pltpu.SemaphoreType.DMA,
            pltpu.SemaphoreType.DMA,
        ],
    )
    def k(fidx_hbm, f_hbm, out_hbm, idx_v, buf0, buf1, f_sh, sg0, sg1, ss0, ss1):
        sid = lax.axis_index("s")
        wid = sid * _NC + lax.axis_index("c")
        base = wid * _PER_W

        @pl.when(sid == 0)
        def _():
            pltpu.sync_copy(f_hbm, f_sh)

        pltpu.sync_copy(fidx_hbm.at[wid], idx_v)
        plsc.subcore_barrier()

        bufs = (buf0, buf1)
        sgs = (sg0, sg1)
        sss = (ss0, ss1)

        def start_gather(h, b):
            pltpu.async_copy(f_sh.at[idx_v.at[h]], bufs[b], sgs[b])

        def wait_gather(b):
            pltpu.make_async_copy(f_sh.at[idx_v.at[0]], bufs[b], sgs[b]).wait()

        def start_store(h, b):
            pltpu.async_copy(bufs[b], out_hbm.at[pl.ds(base + h * _STEP, _STEP)], sss[b])

        def wait_store(b):
            pltpu.make_async_copy(
                bufs[b], out_hbm.at[pl.ds(base, _STEP)], sss[b]
            ).wait()

        start_gather(0, 0)

        def substep(h, b):
            ob = 1 - b

            @pl.when(h >= 1)
            def _():
                wait_store(ob)

            @pl.when(h + 1 < _H)
            def _():
                start_gather(h + 1, ob)

            wait_gather(b)
            start_store(h, b)

        def body(q, carry):
            substep(2 * q, 0)
            substep(2 * q + 1, 1)
            return carry

        lax.fori_loop(0, _H // 2, body, 0)
        wait_store(1)

    return k(fidx3, table2)


def kernel(src, aa_table, pos_table, gamma, beta):
    table = _build_table(aa_table, pos_table, gamma, beta)
    fidx = _build_fidx(src)
    table2 = table.reshape(VOCAB * MAX_POS, HIDDEN)
    fidx3 = fidx.reshape(_NW, _G, _CHUNK)
    out = _sc_gather(fidx3, table2)
    return out.reshape(ROWS, COLS, HIDDEN)


# speedup vs baseline: 33.8235x; 1.0150x over previous
import functools

import jax
import jax.numpy as jnp
from jax import lax
from jax.experimental import pallas as pl
from jax.experimental.pallas import tpu as pltpu
from jax.experimental.pallas import tpu_sc as plsc

VOCAB = 25
MAX_POS = 256
HIDDEN = 128
EPS = 1e-12

ROWS = 4096
COLS = 200
TOKENS = ROWS * COLS


def _prep_body(src_ref, aa_ref, pos_ref, g_ref, b_ref, f_ref, fidx_ref):
    aa = aa_ref[...]
    pos = pos_ref[...]
    e = aa[:, None, :] + pos[None, :, :]
    mean = jnp.mean(e, axis=-1, keepdims=True)
    c = e - mean
    var = jnp.mean(c * c, axis=-1, keepdims=True)
    y = c * lax.rsqrt(var + EPS)
    y = y * g_ref[...][None, None, :] + b_ref[...][None, None, :]
    f_ref[...] = y

    s = src_ref[...]
    m = (s != 0).astype(jnp.int32)
    mb = m.astype(jnp.bfloat16)
    tri = (
        lax.broadcasted_iota(jnp.int32, (COLS, COLS), 0)
        <= lax.broadcasted_iota(jnp.int32, (COLS, COLS), 1)
    ).astype(jnp.bfloat16)
    cs = lax.dot_general(
        mb, tri, (((1,), (0,)), ((), ())),
        preferred_element_type=jnp.float32,
    )
    pid = cs.astype(jnp.int32) * m
    fidx_ref[...] = s * MAX_POS + pid


def _prep(src, aa, pos, gamma, beta):
    return pl.pallas_call(
        _prep_body,
        out_shape=(
            jax.ShapeDtypeStruct((VOCAB, MAX_POS, HIDDEN), jnp.float32),
            jax.ShapeDtypeStruct((ROWS, COLS), jnp.int32),
        ),
    )(src, aa, pos, gamma, beta)


try:
    _INFO = plsc.get_sparse_core_info()
    _NC, _NS = _INFO.num_cores, _INFO.num_subcores
except ValueError:
    _NC, _NS = 2, 16
_NW = _NC * _NS
_CHUNK = 128
_PER_W = TOKENS // _NW
_G = _PER_W // _CHUNK


def _sc_gather(fidx3, table2):
    mesh = plsc.VectorSubcoreMesh(core_axis_name="c", subcore_axis_name="s")

    @functools.partial(
        pl.kernel,
        mesh=mesh,
        out_type=jax.ShapeDtypeStruct((TOKENS, HIDDEN), jnp.float32),
        scratch_types=[
            pltpu.VMEM((_G, _CHUNK), jnp.int32),
            pltpu.VMEM((_CHUNK, HIDDEN), jnp.float32),
            pltpu.VMEM((_CHUNK, HIDDEN), jnp.float32),
            pltpu.VMEM_SHARED((VOCAB * MAX_POS, HIDDEN), jnp.float32),
            pltpu.SemaphoreType.DMA,
            pltpu.SemaphoreType.DMA,
            pltpu.SemaphoreType.DMA,
            pltpu.SemaphoreType.DMA,
        ],
    )
    def k(fidx_hbm, f_hbm, out_hbm, idx_v, buf0, buf1, f_sh, sg0, sg1, ss0, ss1):
        sid = lax.axis_index("s")
        wid = sid * _NC + lax.axis_index("c")
        base = wid * _PER_W

        @pl.when(sid == 0)
        def _():
            pltpu.async_copy(f_hbm, f_sh, sg0)

        pltpu.sync_copy(fidx_hbm.at[wid], idx_v)

        @pl.when(sid == 0)
        def _():
            pltpu.make_async_copy(f_hbm, f_sh, sg0).wait()

        plsc.subcore_barrier()

        bufs = (buf0, buf1)
        sgs = (sg0, sg1)
        sss = (ss0, ss1)

        def start_gather(h, b):
            pltpu.async_copy(f_sh.at[idx_v.at[h]], bufs[b], sgs[b])

        def wait_gather(b):
            pltpu.make_async_copy(f_sh.at[idx_v.at[0]], bufs[b], sgs[b]).wait()

        def start_store(h, b):
            pltpu.async_copy(bufs[b], out_hbm.at[pl.ds(base + h * _CHUNK, _CHUNK)], sss[b])

        def wait_store(b):
            pltpu.make_async_copy(
                bufs[b], out_hbm.at[pl.ds(base, _CHUNK)], sss[b]
            ).wait()

        start_gather(0, 0)

        def substep(h, b):
            ob = 1 - b

            @pl.when(h >= 1)
            def _():
                wait_store(ob)

            @pl.when(h + 1 < _G)
            def _():
                start_gather(h + 1, ob)

            wait_gather(b)
            start_store(h, b)

        def body(q, carry):
            substep(2 * q, 0)
            substep(2 * q + 1, 1)
            return carry

        lax.fori_loop(0, _G // 2, body, 0)
        wait_store(1)

    return k(fidx3, table2)


def kernel(src, aa_table, pos_table, gamma, beta):
    table, fidx = _prep(src, aa_table, pos_table, gamma, beta)
    table2 = table.reshape(VOCAB * MAX_POS, HIDDEN)
    fidx3 = fidx.reshape(_NW, _G, _CHUNK)
    out = _sc_gather(fidx3, table2)
    return out.reshape(ROWS, COLS, HIDDEN)
